# baseline (speedup 1.0000x reference)
import functools

import jax
import jax.numpy as jnp
from jax import lax
from jax.experimental import pallas as pl
from jax.experimental.pallas import tpu as pltpu
from jax.experimental.pallas import tpu_sc as plsc

N_EXPERTS = 8
TOP_K = 2
CHUNK = 1024
N_BUF = 4

_NC, _NS, _L = 2, 16, 16
_NW = _NC * _NS


def _gate_kernel(x_hbm, w_ref, probs_ref, buf, sems):
    n_tokens = x_hbm.shape[0]
    n_chunks = n_tokens // CHUNK
    w = w_ref[...]

    def start_copy(c, slot):
        pltpu.make_async_copy(
            x_hbm.at[pl.ds(c * CHUNK, CHUNK), :],
            buf.at[slot],
            sems.at[slot],
        ).start()

    for c in range(min(N_BUF, n_chunks)):
        start_copy(c, c)
    for c in range(n_chunks):
        slot = c % N_BUF
        pltpu.make_async_copy(
            x_hbm.at[pl.ds(c * CHUNK, CHUNK), :],
            buf.at[slot],
            sems.at[slot],
        ).wait()
        lg = jax.lax.dot_general(
            w, buf[slot], (((1,), (1,)), ((), ())),
            preferred_element_type=jnp.float32,
        )
        m = jnp.max(lg, axis=0, keepdims=True)
        e = jnp.exp(lg - m)
        s = jnp.sum(e, axis=0, keepdims=True)
        probs_ref[:, pl.ds(c * CHUNK, CHUNK)] = e / s
        nxt = c + N_BUF
        if nxt < n_chunks:
            start_copy(nxt, slot)


def _gate(x, w):
    n_tokens, d_model = x.shape
    return pl.pallas_call(
        _gate_kernel,
        in_specs=[
            pl.BlockSpec(memory_space=pltpu.HBM),
            pl.BlockSpec(memory_space=pltpu.VMEM),
        ],
        out_specs=pl.BlockSpec(memory_space=pltpu.VMEM),
        out_shape=jax.ShapeDtypeStruct((N_EXPERTS, n_tokens), jnp.float32),
        scratch_shapes=[
            pltpu.VMEM((N_BUF, CHUNK, 768), jnp.float32),
            pltpu.SemaphoreType.DMA((N_BUF,)),
        ],
    )(x, w)


def _topk_body(probs_hbm, idx_hbm, vals_hbm, pbuf, ibuf, vbuf, sem):
    n_tokens = probs_hbm.shape[1]
    per_w = n_tokens // _NW
    wid = lax.axis_index("s") * _NC + lax.axis_index("c")
    base = wid * per_w
    pltpu.async_copy(probs_hbm.at[:, pl.ds(base, per_w)], pbuf, sem).wait()
    neg_inf = jnp.full((_L,), -jnp.inf, jnp.float32)
    for g in range(per_w // _L):
        sl = pl.ds(g * _L, _L)
        v1 = pbuf[0, sl]
        i1 = jnp.zeros((_L,), jnp.int32)
        for e2 in range(1, N_EXPERTS):
            pe = pbuf[e2, sl]
            m = pe > v1
            v1 = jnp.where(m, pe, v1)
            i1 = jnp.where(m, jnp.full((_L,), e2, jnp.int32), i1)
        v2 = neg_inf
        i2 = jnp.zeros((_L,), jnp.int32)
        for e2 in range(N_EXPERTS):
            pe = jnp.where(i1 == e2, neg_inf, pbuf[e2, sl])
            m = pe > v2
            v2 = jnp.where(m, pe, v2)
            i2 = jnp.where(m, jnp.full((_L,), e2, jnp.int32), i2)
        ibuf[0, sl] = i1
        ibuf[1, sl] = i2
        vbuf[0, sl] = v1
        vbuf[1, sl] = v2
    pltpu.async_copy(ibuf, idx_hbm.at[:, pl.ds(base, per_w)], sem).wait()
    pltpu.async_copy(vbuf, vals_hbm.at[:, pl.ds(base, per_w)], sem).wait()


def _topk_sc(probs_t):
    n_tokens = probs_t.shape[1]
    per_w = n_tokens // _NW
    mesh = plsc.VectorSubcoreMesh(core_axis_name="c", subcore_axis_name="s")
    return pl.kernel(
        _topk_body,
        mesh=mesh,
        out_type=(
            jax.ShapeDtypeStruct((TOP_K, n_tokens), jnp.int32),
            jax.ShapeDtypeStruct((TOP_K, n_tokens), jnp.float32),
        ),
        scratch_types=[
            pltpu.VMEM((N_EXPERTS, per_w), jnp.float32),
            pltpu.VMEM((TOP_K, per_w), jnp.int32),
            pltpu.VMEM((TOP_K, per_w), jnp.float32),
            pltpu.SemaphoreType.DMA,
        ],
    )(probs_t)


@functools.partial(jax.jit, static_argnames=())
def kernel(x, w):
    probs_t = _gate(x, w)
    idx_t, vals_t = _topk_sc(probs_t)
    return (probs_t.T, idx_t.T, vals_t.T)


# speedup vs baseline: 1.5747x; 1.5747x over previous
import functools

import jax
import jax.numpy as jnp
from jax.experimental import pallas as pl
from jax.experimental.pallas import tpu as pltpu

N_EXPERTS = 8
TOP_K = 2
CHUNK = 512
N_BUF = 8


def _chunk_compute(w, x_chunk, probs_ref, idx_ref, vals_ref, base):
    lg = jax.lax.dot_general(
        w, x_chunk, (((1,), (1,)), ((), ())),
        preferred_element_type=jnp.float32,
    )
    m = jnp.max(lg, axis=0, keepdims=True)
    e = jnp.exp(lg - m)
    s = jnp.sum(e, axis=0, keepdims=True)
    p = e / s
    probs_ref[:, pl.ds(base, CHUNK)] = p

    row = jax.lax.broadcasted_iota(jnp.int32, p.shape, 0)
    v1 = jnp.max(p, axis=0, keepdims=True)
    i1 = jnp.min(jnp.where(p == v1, row, N_EXPERTS), axis=0, keepdims=True)
    masked = jnp.where(row == i1, -jnp.inf, p)
    v2 = jnp.max(masked, axis=0, keepdims=True)
    i2 = jnp.min(jnp.where(masked == v2, row, N_EXPERTS), axis=0, keepdims=True)
    idx_ref[:, pl.ds(base, CHUNK)] = jnp.concatenate([i1, i2], axis=0)
    vals_ref[:, pl.ds(base, CHUNK)] = jnp.concatenate([v1, v2], axis=0)


def _router_kernel(x_hbm, w_ref, probs_ref, idx_ref, vals_ref, buf, sems):
    n_tokens = x_hbm.shape[0]
    n_chunks = n_tokens // CHUNK
    w = w_ref[...]

    def start_copy(c, slot):
        pltpu.make_async_copy(
            x_hbm.at[pl.ds(c * CHUNK, CHUNK), :],
            buf.at[slot],
            sems.at[slot],
        ).start()

    for c in range(min(N_BUF, n_chunks)):
        start_copy(c, c)
    for c in range(n_chunks):
        slot = c % N_BUF
        pltpu.make_async_copy(
            x_hbm.at[pl.ds(c * CHUNK, CHUNK), :],
            buf.at[slot],
            sems.at[slot],
        ).wait()
        _chunk_compute(w, buf[slot], probs_ref, idx_ref, vals_ref, c * CHUNK)
        nxt = c + N_BUF
        if nxt < n_chunks:
            start_copy(nxt, slot)


@functools.partial(jax.jit, static_argnames=())
def kernel(x, w):
    n_tokens, d_model = x.shape
    out_shapes = (
        jax.ShapeDtypeStruct((N_EXPERTS, n_tokens), jnp.float32),
        jax.ShapeDtypeStruct((TOP_K, n_tokens), jnp.int32),
        jax.ShapeDtypeStruct((TOP_K, n_tokens), jnp.float32),
    )
    probs_t, idx_t, vals_t = pl.pallas_call(
        _router_kernel,
        in_specs=[
            pl.BlockSpec(memory_space=pltpu.HBM),
            pl.BlockSpec(memory_space=pltpu.VMEM),
        ],
        out_specs=(
            pl.BlockSpec(memory_space=pltpu.VMEM),
            pl.BlockSpec(memory_space=pltpu.VMEM),
            pl.BlockSpec(memory_space=pltpu.VMEM),
        ),
        out_shape=out_shapes,
        scratch_shapes=[
            pltpu.VMEM((N_BUF, CHUNK, 768), jnp.float32),
            pltpu.SemaphoreType.DMA((N_BUF,)),
        ],
    )(x, w)
    return (probs_t.T, idx_t.T, vals_t.T)
